# Initial kernel scaffold; baseline (speedup 1.0000x reference)
import jax
import jax.numpy as jnp
from jax.experimental import pallas as pl


def kernel(x, edge_index, node_type, batch, We1, We2, params):
    raise NotImplementedError("write your pallas kernel here")


# speedup vs baseline: 3.1850x; 3.1850x over previous
import functools

import jax
import jax.numpy as jnp
from jax import lax
from jax.experimental import pallas as pl
from jax.experimental.pallas import tpu as pltpu
from jax.experimental.pallas import tpu_sc as plsc

B = 20
N = 500
NT = B * N
G = 512
NTP = B * G
E = 160000
D = 256
DH = 128
EPS = 1e-5
CHUNK = 128
NCHUNKS = E // CHUNK
NC = 2
NS = 16
STRIPE = NTP // NS
F32 = jnp.float32


def _dot(a, b):
    return jnp.dot(a, b, preferred_element_type=F32)


def _enc_body(x_ref, q_ref, We1_ref, We2_ref, Wz_ref, bz_ref, Wc_ref, bc_ref,
              z_ref, h_ref, hq_s):
    i = pl.program_id(0)

    @pl.when(i == 0)
    def _():
        hq_s[...] = _dot(q_ref[...], Wc_ref[0:D, :])

    he = jax.nn.relu(_dot(x_ref[...], We1_ref[...]))
    m = jnp.sum(he, axis=0, keepdims=True) * (1.0 / N)
    zb = _dot(m, We2_ref[...])
    z_ref[...] = zb[None]
    zc = _dot(zb, Wz_ref[...]) + bz_ref[...]
    h_ref[...] = hq_s[...] + _dot(zc, Wc_ref[D:2 * D, :]) + bc_ref[...]


def _encoder(xp, q, We1, We2, Wz, bz, Wc, bc):
    z3, h0 = pl.pallas_call(
        _enc_body,
        grid=(B,),
        in_specs=[
            pl.BlockSpec((G, D), lambda i: (i, 0)),
            pl.BlockSpec((G, D), lambda i: (0, 0)),
            pl.BlockSpec((D, D), lambda i: (0, 0)),
            pl.BlockSpec((D, D), lambda i: (0, 0)),
            pl.BlockSpec((D, D), lambda i: (0, 0)),
            pl.BlockSpec((1, D), lambda i: (0, 0)),
            pl.BlockSpec((2 * D, D), lambda i: (0, 0)),
            pl.BlockSpec((1, D), lambda i: (0, 0)),
        ],
        out_specs=[
            pl.BlockSpec((1, 1, D), lambda i: (i, 0, 0)),
            pl.BlockSpec((G, D), lambda i: (i, 0)),
        ],
        out_shape=[
            jax.ShapeDtypeStruct((B, 1, D), F32),
            jax.ShapeDtypeStruct((NTP, D), F32),
        ],
        scratch_shapes=[pltpu.VMEM((G, D), F32)],
    )(xp, q, We1, We2, Wz, bz, Wc, bc)
    return z3.reshape(B, D), h0


def _mm_body(h_ref, Wm_ref, bm_ref, t2_ref):
    t = _dot(h_ref[...], Wm_ref[...]) + bm_ref[...]
    t2_ref[0] = t[:, 0:DH]
    t2_ref[1] = t[:, DH:D]


def _transform(h, Wm, bm):
    t2 = pl.pallas_call(
        _mm_body,
        grid=(10,),
        in_specs=[
            pl.BlockSpec((NTP // 10, D), lambda i: (i, 0)),
            pl.BlockSpec((D, D), lambda i: (0, 0)),
            pl.BlockSpec((1, D), lambda i: (0, 0)),
        ],
        out_specs=pl.BlockSpec((2, NTP // 10, DH), lambda i: (0, i, 0)),
        out_shape=jax.ShapeDtypeStruct((2, NTP, DH), F32),
    )(h, Wm, bm)
    return t2.reshape(2 * NTP, DH)


def _spmm(tcat, src1, dst2):
    mesh = plsc.VectorSubcoreMesh(core_axis_name="c", subcore_axis_name="s")

    @functools.partial(
        pl.kernel,
        mesh=mesh,
        out_type=jax.ShapeDtypeStruct((2 * NTP, DH), F32),
        scratch_types=[
            pltpu.VMEM((CHUNK,), jnp.int32),
            pltpu.VMEM((1, CHUNK), jnp.int32),
            pltpu.VMEM((CHUNK, DH), F32),
            pltpu.VMEM_SHARED((NTP, DH), F32),
            pltpu.SemaphoreType.DMA,
        ],
    )
    def k(tcat_hbm, src_hbm, dst_hbm, out_hbm, srcv, dstv, rowbuf, agg_sh, sem):
        ci = lax.axis_index("c")
        s = lax.axis_index("s")
        coff = ci * NTP
        cvec = jnp.full((16,), coff, jnp.int32)

        @pl.loop(0, CHUNK)
        def _(r):
            @pl.loop(0, DH, step=16)
            def _(l):
                rowbuf[pl.ds(r, 1), pl.ds(l, 16)] = jnp.zeros((1, 16), F32)

        for j in range(STRIPE // CHUNK):
            pltpu.sync_copy(rowbuf, agg_sh.at[pl.ds(s * STRIPE + j * CHUNK, CHUNK)])
        plsc.subcore_barrier()

        @pl.loop(0, (NCHUNKS + NS - 1) // NS)
        def _(kk):
            c = s + kk * NS

            @pl.when(c < NCHUNKS)
            def _():
                pltpu.sync_copy(src_hbm.at[pl.ds(c * CHUNK, CHUNK)], srcv)
                pltpu.sync_copy(dst_hbm.at[pl.ds(c, 1)], dstv)

                @pl.loop(0, CHUNK, step=16)
                def _(l):
                    srcv[pl.ds(l, 16)] = srcv[pl.ds(l, 16)] + cvec

                pltpu.async_copy(tcat_hbm.at[srcv], rowbuf, sem).wait()
                pltpu.sync_copy(rowbuf, agg_sh.at[dstv.at[0]], add=True)

        plsc.subcore_barrier()
        pltpu.sync_copy(agg_sh.at[pl.ds(s * STRIPE, STRIPE)],
                        out_hbm.at[pl.ds(coff + s * STRIPE, STRIPE)])

    return k(tcat, src1, dst2)


def _ln_body(h_ref, aL_ref, aR_ref, g_ref, b_ref, o_ref):
    hh = h_ref[...] + jnp.concatenate([aL_ref[...], aR_ref[...]], axis=1)
    mu = jnp.mean(hh, axis=1, keepdims=True)
    d = hh - mu
    var = jnp.mean(d * d, axis=1, keepdims=True)
    o_ref[...] = d * lax.rsqrt(var + EPS) * g_ref[...] + b_ref[...]


def _layernorm(h, aggcat, g, b):
    return pl.pallas_call(
        _ln_body,
        grid=(10,),
        in_specs=[
            pl.BlockSpec((NTP // 10, D), lambda i: (i, 0)),
            pl.BlockSpec((NTP // 10, DH), lambda i: (i, 0)),
            pl.BlockSpec((NTP // 10, DH), lambda i: (i + 10, 0)),
            pl.BlockSpec((1, D), lambda i: (0, 0)),
            pl.BlockSpec((1, D), lambda i: (0, 0)),
        ],
        out_specs=pl.BlockSpec((NTP // 10, D), lambda i: (i, 0)),
        out_shape=jax.ShapeDtypeStruct((NTP, D), F32),
    )(h, aggcat, aggcat, g, b)


def _heads_body(h_ref, q_ref, nt_ref, Wco_ref, bco_ref, Wct_ref, bct_ref,
                Wnt_ref, bnt_ref, Wfr_ref, bfr_ref, Wed_ref, bed_ref,
                Wdir_ref, bdir_ref,
                cen_ref, nto_ref, fr_ref, U_ref, V_ref, qn_s):
    i = pl.program_id(0)
    hb = h_ref[...]

    @pl.when(i == 0)
    def _():
        qn_s[...] = _dot(q_ref[...], Wnt_ref[D:2 * D, :])

    co = _dot(hb, Wco_ref[...]) + bco_ref[...]
    ct = _dot(hb, Wct_ref[...]) + bct_ref[...]
    cen_ref[...] = jnp.where(nt_ref[...] == 0, co, ct)
    nto_ref[...] = _dot(hb, Wnt_ref[0:D, :]) + qn_s[...] + bnt_ref[...]
    fr_ref[...] = _dot(hb, Wfr_ref[...]) + bfr_ref[...]
    zpad = jnp.zeros((G, 12), F32)
    ued = _dot(hb, Wed_ref[0:D, :]) + bed_ref[...]
    udir = _dot(hb, Wdir_ref[0:D, :]) + bdir_ref[...]
    U_ref[...] = jnp.concatenate([ued, udir, zpad], axis=1)
    ved = _dot(hb, Wed_ref[D:2 * D, :])
    vdir = _dot(hb, Wdir_ref[D:2 * D, :])
    V_ref[...] = jnp.concatenate([ved, vdir, zpad], axis=1)


def _heads(h, q, ntp, p):
    full = lambda shape: pl.BlockSpec(shape, lambda i: (0, 0))
    return pl.pallas_call(
        _heads_body,
        grid=(B,),
        in_specs=[
            pl.BlockSpec((G, D), lambda i: (i, 0)),
            pl.BlockSpec((G, D), lambda i: (0, 0)),
            pl.BlockSpec((G, 1), lambda i: (i, 0)),
            full((D, 3)), full((1, 3)),
            full((D, 3)), full((1, 3)),
            full((2 * D, 4)), full((1, 4)),
            full((D, 1)), full((1, 1)),
            full((2 * D, 1)), full((1, 1)),
            full((2 * D, 3)), full((1, 3)),
        ],
        out_specs=[
            pl.BlockSpec((G, 3), lambda i: (i, 0)),
            pl.BlockSpec((G, 4), lambda i: (i, 0)),
            pl.BlockSpec((G, 1), lambda i: (i, 0)),
            pl.BlockSpec((G, 16), lambda i: (i, 0)),
            pl.BlockSpec((G, 16), lambda i: (i, 0)),
        ],
        out_shape=[
            jax.ShapeDtypeStruct((NTP, 3), F32),
            jax.ShapeDtypeStruct((NTP, 4), F32),
            jax.ShapeDtypeStruct((NTP, 1), F32),
            jax.ShapeDtypeStruct((NTP, 16), F32),
            jax.ShapeDtypeStruct((NTP, 16), F32),
        ],
        scratch_shapes=[pltpu.VMEM((G, 4), F32)],
    )(h, q, ntp,
      p['Wco'], p['bco'].reshape(1, 3), p['Wct'], p['bct'].reshape(1, 3),
      p['Wnt'], p['bnt'].reshape(1, 4), p['Wfr'], p['bfr'].reshape(1, 1),
      p['Wed'], p['bed'].reshape(1, 1), p['Wdir'], p['bdir'].reshape(1, 3))


def _edge_gather(U16, V16, src1, dst1):
    mesh = plsc.VectorSubcoreMesh(core_axis_name="c", subcore_axis_name="s")

    @functools.partial(
        pl.kernel,
        mesh=mesh,
        compiler_params=pltpu.CompilerParams(use_tc_tiling_on_sc=False),
        out_type=(jax.ShapeDtypeStruct((E, 16), F32),
                  jax.ShapeDtypeStruct((E, 16), F32)),
        scratch_types=[
            pltpu.VMEM((CHUNK,), jnp.int32),
            pltpu.VMEM((CHUNK,), jnp.int32),
            pltpu.VMEM((CHUNK, 16), F32),
            pltpu.VMEM((CHUNK, 16), F32),
            pltpu.SemaphoreType.DMA,
        ],
    )
    def k(U_hbm, V_hbm, src_hbm, dst_hbm, gu_hbm, gv_hbm,
          srcv, dstv, bufU, bufV, sem):
        ci = lax.axis_index("c")
        s = lax.axis_index("s")
        w = s * NC + ci
        nw = NC * NS

        @pl.loop(0, (NCHUNKS + NC * NS - 1) // (NC * NS))
        def _(kk):
            c = w + kk * nw

            @pl.when(c < NCHUNKS)
            def _():
                pltpu.sync_copy(src_hbm.at[pl.ds(c * CHUNK, CHUNK)], srcv)
                pltpu.sync_copy(dst_hbm.at[pl.ds(c * CHUNK, CHUNK)], dstv)
                pltpu.async_copy(U_hbm.at[srcv], bufU, sem).wait()
                pltpu.async_copy(V_hbm.at[dstv], bufV, sem).wait()
                pltpu.sync_copy(bufU, gu_hbm.at[pl.ds(c * CHUNK, CHUNK)])
                pltpu.sync_copy(bufV, gv_hbm.at[pl.ds(c * CHUNK, CHUNK)])

    return k(U16, V16, src1, dst1)


def _eadd_body(gu_ref, gv_ref, dist_ref, dir_ref):
    ssum = gu_ref[...] + gv_ref[...]
    dist_ref[...] = ssum[:, 0:1]
    dir_ref[...] = ssum[:, 1:4]


def _edge_add(gu, gv):
    RB = 2000
    return pl.pallas_call(
        _eadd_body,
        grid=(E // RB,),
        in_specs=[
            pl.BlockSpec((RB, 16), lambda i: (i, 0)),
            pl.BlockSpec((RB, 16), lambda i: (i, 0)),
        ],
        out_specs=[
            pl.BlockSpec((RB, 1), lambda i: (i, 0)),
            pl.BlockSpec((RB, 3), lambda i: (i, 0)),
        ],
        out_shape=[
            jax.ShapeDtypeStruct((E, 1), F32),
            jax.ShapeDtypeStruct((E, 3), F32),
        ],
    )(gu, gv)


def _unpad(a):
    return a.reshape(B, G, -1)[:, :N].reshape(NT, -1)


def kernel(x, edge_index, node_type, batch, We1, We2, params):
    p = params
    xp = jnp.pad(x.reshape(B, N, D), ((0, 0), (0, G - N), (0, 0))).reshape(NTP, D)
    src = edge_index[0].astype(jnp.int32)
    dst = edge_index[1].astype(jnp.int32)
    srcP = src + (G - N) * (src // N)
    dstP = dst + (G - N) * (dst // N)
    dstP2 = dstP.reshape(NCHUNKS, CHUNK)
    ntp = jnp.pad(node_type.astype(jnp.int32).reshape(B, N),
                  ((0, 0), (0, G - N))).reshape(NTP, 1)
    q = p['node_queries']

    z, h = _encoder(xp, q, We1, We2, p['Wz'], p['bz'].reshape(1, D),
                    p['Wc'], p['bc'].reshape(1, D))

    for Wm, bm, g, bn in ((p['Wm1'], p['bm1'], p['g1'], p['b1']),
                          (p['Wm2'], p['bm2'], p['g2'], p['b2'])):
        tcat = _transform(h, Wm, bm.reshape(1, D))
        aggcat = _spmm(tcat, srcP, dstP2)
        h = _layernorm(h, aggcat, g.reshape(1, D), bn.reshape(1, D))

    cen, nto, fr, U16, V16 = _heads(h, q, ntp, p)
    gu, gv = _edge_gather(U16, V16, srcP, dstP)
    dist, edir = _edge_add(gu, gv)

    recon = {'centroids': _unpad(cen),
             'node_types': _unpad(nto),
             'frequency': _unpad(fr),
             'edge_dist': dist,
             'edge_dir': edir}
    return (z, recon)


# speedup vs baseline: 4.1410x; 1.3002x over previous
import functools

import jax
import jax.numpy as jnp
from jax import lax
from jax.experimental import pallas as pl
from jax.experimental.pallas import tpu as pltpu
from jax.experimental.pallas import tpu_sc as plsc

B = 20
N = 500
NT = B * N
G = 512
NTP = B * G
E = 160000
D = 256
DH = 128
EPS = 1e-5
CHUNK = 128
NCHUNKS = E // CHUNK
NC = 2
NS = 16
STRIPE = NTP // NS
F32 = jnp.float32


def _dot(a, b):
    return jnp.dot(a, b, preferred_element_type=F32)


def _enc_body(x_ref, q_ref, We1_ref, We2_ref, Wz_ref, bz_ref, Wc_ref, bc_ref,
              z_ref, h_ref, hq_s):
    i = pl.program_id(0)

    @pl.when(i == 0)
    def _():
        hq_s[...] = _dot(q_ref[...], Wc_ref[0:D, :])

    he = jax.nn.relu(_dot(x_ref[...], We1_ref[...]))
    m = jnp.sum(he, axis=0, keepdims=True) * (1.0 / N)
    zb = _dot(m, We2_ref[...])
    z_ref[...] = zb[None]
    zc = _dot(zb, Wz_ref[...]) + bz_ref[...]
    h_ref[...] = hq_s[...] + _dot(zc, Wc_ref[D:2 * D, :]) + bc_ref[...]


def _encoder(xp, q, We1, We2, Wz, bz, Wc, bc):
    z3, h0 = pl.pallas_call(
        _enc_body,
        grid=(B,),
        in_specs=[
            pl.BlockSpec((G, D), lambda i: (i, 0)),
            pl.BlockSpec((G, D), lambda i: (0, 0)),
            pl.BlockSpec((D, D), lambda i: (0, 0)),
            pl.BlockSpec((D, D), lambda i: (0, 0)),
            pl.BlockSpec((D, D), lambda i: (0, 0)),
            pl.BlockSpec((1, D), lambda i: (0, 0)),
            pl.BlockSpec((2 * D, D), lambda i: (0, 0)),
            pl.BlockSpec((1, D), lambda i: (0, 0)),
        ],
        out_specs=[
            pl.BlockSpec((1, 1, D), lambda i: (i, 0, 0)),
            pl.BlockSpec((G, D), lambda i: (i, 0)),
        ],
        out_shape=[
            jax.ShapeDtypeStruct((B, 1, D), F32),
            jax.ShapeDtypeStruct((NTP, D), F32),
        ],
        scratch_shapes=[pltpu.VMEM((G, D), F32)],
    )(xp, q, We1, We2, Wz, bz, Wc, bc)
    return z3.reshape(B, D), h0


def _mm_body(h_ref, Wm_ref, bm_ref, t2_ref):
    t = _dot(h_ref[...], Wm_ref[...]) + bm_ref[...]
    t2_ref[0] = t[:, 0:DH]
    t2_ref[1] = t[:, DH:D]


def _transform(h, Wm, bm):
    t2 = pl.pallas_call(
        _mm_body,
        grid=(10,),
        in_specs=[
            pl.BlockSpec((NTP // 10, D), lambda i: (i, 0)),
            pl.BlockSpec((D, D), lambda i: (0, 0)),
            pl.BlockSpec((1, D), lambda i: (0, 0)),
        ],
        out_specs=pl.BlockSpec((2, NTP // 10, DH), lambda i: (0, i, 0)),
        out_shape=jax.ShapeDtypeStruct((2, NTP, DH), F32),
    )(h, Wm, bm)
    return t2.reshape(2 * NTP, DH)


NBUF = 4
NB_S = 2
KMAX = 80


def _spmm(tcat, src2, dst2):
    mesh = plsc.VectorSubcoreMesh(core_axis_name="c", subcore_axis_name="s")

    @functools.partial(
        pl.kernel,
        mesh=mesh,
        out_type=jax.ShapeDtypeStruct((2 * NTP, DH), F32),
        scratch_types=[
            pltpu.VMEM((NB_S, CHUNK), jnp.int32),
            pltpu.VMEM((NB_S, CHUNK), jnp.int32),
            [pltpu.VMEM((CHUNK, DH), F32)] * NB_S,
            pltpu.VMEM_SHARED((NTP, DH), F32),
            [pltpu.SemaphoreType.DMA] * NB_S,
        ],
    )
    def k(tcat_hbm, src_hbm, dst_hbm, out_hbm, srcv, dstv, rowbuf,
          agg_sh, sems):
        ci = lax.axis_index("c")
        s = lax.axis_index("s")
        coff = ci * NTP
        cvec = jnp.full((1, 16), coff, jnp.int32)

        @pl.loop(0, CHUNK)
        def _(r):
            @pl.loop(0, DH, step=16)
            def _(l):
                rowbuf[0][pl.ds(r, 1), pl.ds(l, 16)] = jnp.zeros((1, 16), F32)

        for j in range(STRIPE // CHUNK):
            pltpu.sync_copy(rowbuf[0],
                            agg_sh.at[pl.ds(s * STRIPE + j * CHUNK, CHUNK)])
        plsc.subcore_barrier()

        @pl.loop(0, KMAX, step=NB_S)
        def _(kk):
            for j in range(NB_S):
                c = (kk + j) * NS + s

                @pl.when(c < NCHUNKS)
                def _(j=j, c=c):
                    pltpu.async_copy(src_hbm.at[pl.ds(c, 1)],
                                     srcv.at[pl.ds(j, 1)], sems[j])
                    pltpu.async_copy(dst_hbm.at[pl.ds(c, 1)],
                                     dstv.at[pl.ds(j, 1)], sems[j])

            for j in range(NB_S):
                c = (kk + j) * NS + s

                @pl.when(c < NCHUNKS)
                def _(j=j, c=c):
                    pltpu.make_async_copy(src_hbm.at[pl.ds(c, 1)],
                                          srcv.at[pl.ds(j, 1)], sems[j]).wait()
                    pltpu.make_async_copy(dst_hbm.at[pl.ds(c, 1)],
                                          dstv.at[pl.ds(j, 1)], sems[j]).wait()

                    @pl.loop(0, CHUNK, step=16)
                    def _(l):
                        srcv[pl.ds(j, 1), pl.ds(l, 16)] = (
                            srcv[pl.ds(j, 1), pl.ds(l, 16)] + cvec)
                    pltpu.async_copy(tcat_hbm.at[srcv.at[j]],
                                     rowbuf[j], sems[j])

            for j in range(NB_S):
                c = (kk + j) * NS + s

                @pl.when(c < NCHUNKS)
                def _(j=j):
                    pltpu.make_async_copy(tcat_hbm.at[srcv.at[j]],
                                          rowbuf[j], sems[j]).wait()
                    pltpu.async_copy(rowbuf[j], agg_sh.at[dstv.at[j]],
                                     sems[j], add=True)

            for j in range(NB_S):
                c = (kk + j) * NS + s

                @pl.when(c < NCHUNKS)
                def _(j=j):
                    pltpu.make_async_copy(rowbuf[j], agg_sh.at[dstv.at[j]],
                                          sems[j]).wait()

        plsc.subcore_barrier()
        pltpu.sync_copy(agg_sh.at[pl.ds(s * STRIPE, STRIPE)],
                        out_hbm.at[pl.ds(coff + s * STRIPE, STRIPE)])

    return k(tcat, src2, dst2)


def _ln_body(h_ref, aL_ref, aR_ref, g_ref, b_ref, o_ref):
    hh = h_ref[...] + jnp.concatenate([aL_ref[...], aR_ref[...]], axis=1)
    mu = jnp.mean(hh, axis=1, keepdims=True)
    d = hh - mu
    var = jnp.mean(d * d, axis=1, keepdims=True)
    o_ref[...] = d * lax.rsqrt(var + EPS) * g_ref[...] + b_ref[...]


def _layernorm(h, aggcat, g, b):
    return pl.pallas_call(
        _ln_body,
        grid=(10,),
        in_specs=[
            pl.BlockSpec((NTP // 10, D), lambda i: (i, 0)),
            pl.BlockSpec((NTP // 10, DH), lambda i: (i, 0)),
            pl.BlockSpec((NTP // 10, DH), lambda i: (i + 10, 0)),
            pl.BlockSpec((1, D), lambda i: (0, 0)),
            pl.BlockSpec((1, D), lambda i: (0, 0)),
        ],
        out_specs=pl.BlockSpec((NTP // 10, D), lambda i: (i, 0)),
        out_shape=jax.ShapeDtypeStruct((NTP, D), F32),
    )(h, aggcat, aggcat, g, b)


def _heads_body(h_ref, q_ref, nt_ref, Wco_ref, bco_ref, Wct_ref, bct_ref,
                Wnt_ref, bnt_ref, Wfr_ref, bfr_ref, Wed_ref, bed_ref,
                Wdir_ref, bdir_ref,
                cen_ref, nto_ref, fr_ref, U_ref, V_ref, qn_s):
    i = pl.program_id(0)
    hb = h_ref[...]

    @pl.when(i == 0)
    def _():
        qn_s[...] = _dot(q_ref[...], Wnt_ref[D:2 * D, :])

    co = _dot(hb, Wco_ref[...]) + bco_ref[...]
    ct = _dot(hb, Wct_ref[...]) + bct_ref[...]
    cen_ref[...] = jnp.where(nt_ref[...] == 0, co, ct)
    nto_ref[...] = _dot(hb, Wnt_ref[0:D, :]) + qn_s[...] + bnt_ref[...]
    fr_ref[...] = _dot(hb, Wfr_ref[...]) + bfr_ref[...]
    zpad = jnp.zeros((G, 12), F32)
    ued = _dot(hb, Wed_ref[0:D, :]) + bed_ref[...]
    udir = _dot(hb, Wdir_ref[0:D, :]) + bdir_ref[...]
    U_ref[...] = jnp.concatenate([ued, udir, zpad], axis=1)
    ved = _dot(hb, Wed_ref[D:2 * D, :])
    vdir = _dot(hb, Wdir_ref[D:2 * D, :])
    V_ref[...] = jnp.concatenate([ved, vdir, zpad], axis=1)


def _heads(h, q, ntp, p):
    full = lambda shape: pl.BlockSpec(shape, lambda i: (0, 0))
    return pl.pallas_call(
        _heads_body,
        grid=(B,),
        in_specs=[
            pl.BlockSpec((G, D), lambda i: (i, 0)),
            pl.BlockSpec((G, D), lambda i: (0, 0)),
            pl.BlockSpec((G, 1), lambda i: (i, 0)),
            full((D, 3)), full((1, 3)),
            full((D, 3)), full((1, 3)),
            full((2 * D, 4)), full((1, 4)),
            full((D, 1)), full((1, 1)),
            full((2 * D, 1)), full((1, 1)),
            full((2 * D, 3)), full((1, 3)),
        ],
        out_specs=[
            pl.BlockSpec((G, 3), lambda i: (i, 0)),
            pl.BlockSpec((G, 4), lambda i: (i, 0)),
            pl.BlockSpec((G, 1), lambda i: (i, 0)),
            pl.BlockSpec((G, 16), lambda i: (i, 0)),
            pl.BlockSpec((G, 16), lambda i: (i, 0)),
        ],
        out_shape=[
            jax.ShapeDtypeStruct((NTP, 3), F32),
            jax.ShapeDtypeStruct((NTP, 4), F32),
            jax.ShapeDtypeStruct((NTP, 1), F32),
            jax.ShapeDtypeStruct((NTP, 16), F32),
            jax.ShapeDtypeStruct((NTP, 16), F32),
        ],
        scratch_shapes=[pltpu.VMEM((G, 4), F32)],
    )(h, q, ntp,
      p['Wco'], p['bco'].reshape(1, 3), p['Wct'], p['bct'].reshape(1, 3),
      p['Wnt'], p['bnt'].reshape(1, 4), p['Wfr'], p['bfr'].reshape(1, 1),
      p['Wed'], p['bed'].reshape(1, 1), p['Wdir'], p['bdir'].reshape(1, 3))


def _edge_gather(U16, V16, src2, dst2):
    mesh = plsc.VectorSubcoreMesh(core_axis_name="c", subcore_axis_name="s")

    @functools.partial(
        pl.kernel,
        mesh=mesh,
        compiler_params=pltpu.CompilerParams(use_tc_tiling_on_sc=False),
        out_type=(jax.ShapeDtypeStruct((E, 16), F32),
                  jax.ShapeDtypeStruct((E, 16), F32)),
        scratch_types=[
            pltpu.VMEM((NBUF, CHUNK), jnp.int32),
            pltpu.VMEM((NBUF, CHUNK), jnp.int32),
            [pltpu.VMEM((CHUNK, 16), F32)] * NBUF,
            [pltpu.VMEM((CHUNK, 16), F32)] * NBUF,
            [pltpu.SemaphoreType.DMA] * NBUF,
        ],
    )
    def k(U_hbm, V_hbm, src_hbm, dst_hbm, gu_hbm, gv_hbm,
          srcv, dstv, bufU, bufV, sems):
        ci = lax.axis_index("c")
        s = lax.axis_index("s")
        w = s * NC + ci
        nw = NC * NS
        kmax = -(-NCHUNKS // nw)
        kup = -(-kmax // NBUF) * NBUF

        @pl.loop(0, kup, step=NBUF)
        def _(kk):
            for j in range(NBUF):
                c = w + (kk + j) * nw

                @pl.when(c < NCHUNKS)
                def _(j=j, c=c):
                    pltpu.async_copy(src_hbm.at[pl.ds(c, 1)],
                                     srcv.at[pl.ds(j, 1)], sems[j])
                    pltpu.async_copy(dst_hbm.at[pl.ds(c, 1)],
                                     dstv.at[pl.ds(j, 1)], sems[j])

            for j in range(NBUF):
                c = w + (kk + j) * nw

                @pl.when(c < NCHUNKS)
                def _(j=j, c=c):
                    pltpu.make_async_copy(src_hbm.at[pl.ds(c, 1)],
                                          srcv.at[pl.ds(j, 1)], sems[j]).wait()
                    pltpu.make_async_copy(dst_hbm.at[pl.ds(c, 1)],
                                          dstv.at[pl.ds(j, 1)], sems[j]).wait()
                    pltpu.async_copy(U_hbm.at[srcv.at[j]], bufU[j], sems[j])
                    pltpu.async_copy(V_hbm.at[dstv.at[j]], bufV[j], sems[j])

            for j in range(NBUF):
                c = w + (kk + j) * nw

                @pl.when(c < NCHUNKS)
                def _(j=j, c=c):
                    pltpu.make_async_copy(U_hbm.at[srcv.at[j]],
                                          bufU[j], sems[j]).wait()
                    pltpu.make_async_copy(V_hbm.at[dstv.at[j]],
                                          bufV[j], sems[j]).wait()
                    pltpu.async_copy(bufU[j],
                                     gu_hbm.at[pl.ds(c * CHUNK, CHUNK)],
                                     sems[j])
                    pltpu.async_copy(bufV[j],
                                     gv_hbm.at[pl.ds(c * CHUNK, CHUNK)],
                                     sems[j])

            for j in range(NBUF):
                c = w + (kk + j) * nw

                @pl.when(c < NCHUNKS)
                def _(j=j, c=c):
                    pltpu.make_async_copy(bufU[j],
                                          gu_hbm.at[pl.ds(c * CHUNK, CHUNK)],
                                          sems[j]).wait()
                    pltpu.make_async_copy(bufV[j],
                                          gv_hbm.at[pl.ds(c * CHUNK, CHUNK)],
                                          sems[j]).wait()

    return k(U16, V16, src2, dst2)


def _eadd_body(gu_ref, gv_ref, dist_ref, dir_ref):
    ssum = gu_ref[...] + gv_ref[...]
    dist_ref[...] = ssum[:, 0:1]
    dir_ref[...] = ssum[:, 1:4]


def _edge_add(gu, gv):
    RB = 2000
    return pl.pallas_call(
        _eadd_body,
        grid=(E // RB,),
        in_specs=[
            pl.BlockSpec((RB, 16), lambda i: (i, 0)),
            pl.BlockSpec((RB, 16), lambda i: (i, 0)),
        ],
        out_specs=[
            pl.BlockSpec((RB, 1), lambda i: (i, 0)),
            pl.BlockSpec((RB, 3), lambda i: (i, 0)),
        ],
        out_shape=[
            jax.ShapeDtypeStruct((E, 1), F32),
            jax.ShapeDtypeStruct((E, 3), F32),
        ],
    )(gu, gv)


def _unpad(a):
    return a.reshape(B, G, -1)[:, :N].reshape(NT, -1)


def kernel(x, edge_index, node_type, batch, We1, We2, params):
    p = params
    xp = jnp.pad(x.reshape(B, N, D), ((0, 0), (0, G - N), (0, 0))).reshape(NTP, D)
    src = edge_index[0].astype(jnp.int32)
    dst = edge_index[1].astype(jnp.int32)
    srcP2 = (src + (G - N) * (src // N)).reshape(NCHUNKS, CHUNK)
    dstP2 = (dst + (G - N) * (dst // N)).reshape(NCHUNKS, CHUNK)
    ntp = jnp.pad(node_type.astype(jnp.int32).reshape(B, N),
                  ((0, 0), (0, G - N))).reshape(NTP, 1)
    q = p['node_queries']

    z, h = _encoder(xp, q, We1, We2, p['Wz'], p['bz'].reshape(1, D),
                    p['Wc'], p['bc'].reshape(1, D))

    for Wm, bm, g, bn in ((p['Wm1'], p['bm1'], p['g1'], p['b1']),
                          (p['Wm2'], p['bm2'], p['g2'], p['b2'])):
        tcat = _transform(h, Wm, bm.reshape(1, D))
        aggcat = _spmm(tcat, srcP2, dstP2)
        h = _layernorm(h, aggcat, g.reshape(1, D), bn.reshape(1, D))

    cen, nto, fr, U16, V16 = _heads(h, q, ntp, p)
    gu, gv = _edge_gather(U16, V16, srcP2, dstP2)
    dist, edir = _edge_add(gu, gv)

    recon = {'centroids': _unpad(cen),
             'node_types': _unpad(nto),
             'frequency': _unpad(fr),
             'edge_dist': dist,
             'edge_dir': edir}
    return (z, recon)
